# baseline (speedup 1.0000x reference)
import functools

import jax
import jax.numpy as jnp
from jax import lax
from jax.experimental import pallas as pl
from jax.experimental.pallas import tpu as pltpu
from jax.experimental.pallas import tpu_sc as plsc

VOCAB = 100000
EMB = 128
NUM_LABELS = 100000
CTX = 50
BATCH = 1024

NC = 2
NS = 16
NW = NC * NS
BPW = BATCH // NW
CHUNK_B = 2
NCHUNK = BPW // CHUNK_B
ROWS = CTX * CHUNK_B

LANES = 16

BN = 5000
NBLK = NUM_LABELS // BN


UNROLL = 10


def _sc_gather_sum(idx_grp, table):
    mesh = plsc.VectorSubcoreMesh(core_axis_name="c", subcore_axis_name="s")

    @functools.partial(
        pl.kernel,
        out_type=jax.ShapeDtypeStruct((BATCH, EMB), jnp.float32),
        mesh=mesh,
        scratch_types=[
            pltpu.VMEM((NCHUNK, ROWS), jnp.int32),
            pltpu.VMEM((ROWS, EMB), jnp.float32),
            pltpu.VMEM((ROWS, EMB), jnp.float32),
            pltpu.VMEM((ROWS, EMB), jnp.float32),
            pltpu.VMEM((ROWS, EMB), jnp.float32),
            pltpu.VMEM((BPW, EMB), jnp.float32),
            pltpu.SemaphoreType.DMA,
            pltpu.SemaphoreType.DMA,
            pltpu.SemaphoreType.DMA,
            pltpu.SemaphoreType.DMA,
        ],
    )
    def k(idx_hbm, table_hbm, out_hbm, idx_v, rows_a, rows_b, rows_c,
          rows_d, acc_v, sem_a, sem_b, sem_c, sem_d):
        wid = lax.axis_index("s") * NC + lax.axis_index("c")
        pltpu.sync_copy(idx_hbm.at[wid], idx_v)

        def reduce_chunk(rows_v, g):
            NJ = EMB // LANES
            for e in range(CHUNK_B):
                def red(kk, accs, _e=e):
                    for u in range(UNROLL):
                        r = _e * CTX + kk * UNROLL + u
                        accs = tuple(
                            accs[j] + rows_v[r, pl.ds(j * LANES, LANES)]
                            for j in range(NJ))
                    return accs
                accs = lax.fori_loop(
                    0, CTX // UNROLL, red,
                    tuple(jnp.zeros((LANES,), jnp.float32)
                          for _ in range(NJ)))
                for j in range(NJ):
                    acc_v[g * CHUNK_B + e, pl.ds(j * LANES, LANES)] = accs[j]

        bufs = (rows_a, rows_b, rows_c, rows_d)
        sems = (sem_a, sem_b, sem_c, sem_d)

        def start(slot, g):
            pltpu.make_async_copy(table_hbm.at[idx_v.at[g]], bufs[slot],
                                  sems[slot]).start()

        def wait(slot):
            pltpu.make_async_copy(table_hbm.at[idx_v.at[0]], bufs[slot],
                                  sems[slot]).wait()

        start(0, 0)
        start(1, 1)

        def quad(q, carry):
            g = 4 * q
            for s in range(4):
                wait(s)
                start((s + 2) % 4, lax.rem(g + s + 2, NCHUNK))
                reduce_chunk(bufs[s], g + s)
            return carry

        lax.fori_loop(0, NCHUNK // 4, quad, 0)
        wait(0)
        wait(1)
        pltpu.sync_copy(acc_v, out_hbm.at[pl.ds(wid * BPW, BPW)])

    return k(idx_grp, table)


def _tc_body(emb_ref, w_ref, o_ref):
    emb = emb_ref[...].astype(jnp.bfloat16)
    w = w_ref[...].astype(jnp.bfloat16)
    x = lax.dot_general(w, emb, (((1,), (1,)), ((), ())),
                        preferred_element_type=jnp.float32)
    y = x - jnp.max(x, axis=1, keepdims=True)
    lse = jnp.log(jnp.sum(jnp.exp(y), axis=1, keepdims=True))
    o_ref[...] = y - lse


def _tc_classifier(emb_sum, W, interpret=False):
    out_t = pl.pallas_call(
        _tc_body,
        grid=(NBLK,),
        in_specs=[
            pl.BlockSpec((BATCH, EMB), lambda i: (0, 0)),
            pl.BlockSpec((BN, EMB), lambda i: (i, 0)),
        ],
        out_specs=pl.BlockSpec((BN, BATCH), lambda i: (i, 0)),
        out_shape=jax.ShapeDtypeStruct((NUM_LABELS, BATCH), jnp.float32),
        interpret=interpret,
    )(emb_sum, W)
    return out_t.T


def kernel(inputs, emb_table, W, b):
    del b
    idx = inputs.astype(jnp.int32).T.reshape(NW, NCHUNK, ROWS)
    emb_sum = _sc_gather_sum(idx, emb_table)
    return _tc_classifier(emb_sum, W)


# speedup vs baseline: 1.0071x; 1.0071x over previous
import functools

import jax
import jax.numpy as jnp
from jax import lax
from jax.experimental import pallas as pl
from jax.experimental.pallas import tpu as pltpu
from jax.experimental.pallas import tpu_sc as plsc

VOCAB = 100000
EMB = 128
NUM_LABELS = 100000
CTX = 50
BATCH = 1024

NC = 2
NS = 16
NW = NC * NS
BPW = BATCH // NW
CHUNK_B = 2
NCHUNK = BPW // CHUNK_B
ROWS = CTX * CHUNK_B

LANES = 16

BN = 5000
NBLK = NUM_LABELS // BN


UNROLL = 10


def _sc_gather_sum(idx_grp, table):
    mesh = plsc.VectorSubcoreMesh(core_axis_name="c", subcore_axis_name="s")

    @functools.partial(
        pl.kernel,
        out_type=jax.ShapeDtypeStruct((BATCH, EMB), jnp.float32),
        mesh=mesh,
        scratch_types=[
            pltpu.VMEM((NCHUNK, ROWS), jnp.int32),
            pltpu.VMEM((ROWS, EMB), jnp.float32),
            pltpu.VMEM((ROWS, EMB), jnp.float32),
            pltpu.VMEM((ROWS, EMB), jnp.float32),
            pltpu.VMEM((ROWS, EMB), jnp.float32),
            pltpu.VMEM((BPW, EMB), jnp.float32),
            pltpu.SemaphoreType.DMA,
            pltpu.SemaphoreType.DMA,
            pltpu.SemaphoreType.DMA,
            pltpu.SemaphoreType.DMA,
        ],
    )
    def k(idx_hbm, table_hbm, out_hbm, idx_v, rows_a, rows_b, rows_c,
          rows_d, acc_v, sem_a, sem_b, sem_c, sem_d):
        wid = lax.axis_index("s") * NC + lax.axis_index("c")
        pltpu.sync_copy(idx_hbm.at[wid], idx_v)

        def reduce_chunk(rows_v, g):
            NJ = EMB // LANES
            for e in range(CHUNK_B):
                def red(kk, accs, _e=e):
                    for u in range(UNROLL):
                        r = _e * CTX + kk * UNROLL + u
                        accs = tuple(
                            accs[j] + rows_v[r, pl.ds(j * LANES, LANES)]
                            for j in range(NJ))
                    return accs
                accs = lax.fori_loop(
                    0, CTX // UNROLL, red,
                    tuple(jnp.zeros((LANES,), jnp.float32)
                          for _ in range(NJ)))
                for j in range(NJ):
                    acc_v[g * CHUNK_B + e, pl.ds(j * LANES, LANES)] = accs[j]

        bufs = (rows_a, rows_b, rows_c, rows_d)
        sems = (sem_a, sem_b, sem_c, sem_d)

        def start(slot, g):
            pltpu.make_async_copy(table_hbm.at[idx_v.at[g]], bufs[slot],
                                  sems[slot]).start()

        def wait(slot):
            pltpu.make_async_copy(table_hbm.at[idx_v.at[0]], bufs[slot],
                                  sems[slot]).wait()

        start(0, 0)
        start(1, 1)
        start(2, 2)

        def quad(q, carry):
            g = 4 * q
            for s in range(4):
                wait(s)
                start((s + 3) % 4, lax.rem(g + s + 3, NCHUNK))
                reduce_chunk(bufs[s], g + s)
            return carry

        lax.fori_loop(0, NCHUNK // 4, quad, 0)
        wait(0)
        wait(1)
        wait(2)
        pltpu.sync_copy(acc_v, out_hbm.at[pl.ds(wid * BPW, BPW)])

    return k(idx_grp, table)


def _tc_body(emb_ref, w_ref, o_ref):
    emb = emb_ref[...].astype(jnp.bfloat16)
    w = w_ref[...].astype(jnp.bfloat16)
    x = lax.dot_general(w, emb, (((1,), (1,)), ((), ())),
                        preferred_element_type=jnp.float32)
    y = x - jnp.max(x, axis=1, keepdims=True)
    lse = jnp.log(jnp.sum(jnp.exp(y), axis=1, keepdims=True))
    o_ref[...] = y - lse


def _tc_classifier(emb_sum, W, interpret=False):
    out_t = pl.pallas_call(
        _tc_body,
        grid=(NBLK,),
        in_specs=[
            pl.BlockSpec((BATCH, EMB), lambda i: (0, 0)),
            pl.BlockSpec((BN, EMB), lambda i: (i, 0)),
        ],
        out_specs=pl.BlockSpec((BN, BATCH), lambda i: (i, 0)),
        out_shape=jax.ShapeDtypeStruct((NUM_LABELS, BATCH), jnp.float32),
        interpret=interpret,
    )(emb_sum, W)
    return out_t.T


def kernel(inputs, emb_table, W, b):
    del b
    idx = inputs.astype(jnp.int32).T.reshape(NW, NCHUNK, ROWS)
    emb_sum = _sc_gather_sum(idx, emb_table)
    return _tc_classifier(emb_sum, W)
